# baseline (speedup 1.0000x reference)
import functools

import jax
import jax.numpy as jnp
from jax.experimental import pallas as pl
from jax.experimental.pallas import tpu as pltpu

_K = 8192
_D = 128
_BETA = 0.25
_TEMP = 0.1
_NB = 512


def _vq_body(z_ref, znrm_ref, w_ref, wnrm_ref,
             zq_ref, com_ref, ppl_ref, ent_ref,
             wn_ref, psum_ref, cnt_ref, com_acc, n_rows, rb):
    r = pl.program_id(0)

    @pl.when(r == 0)
    def _init():
        wn_ref[...] = w_ref[...] / wnrm_ref[...]
        psum_ref[...] = jnp.zeros_like(psum_ref)
        cnt_ref[...] = jnp.zeros_like(cnt_ref)
        com_acc[...] = jnp.zeros_like(com_acc)

    z = z_ref[...]
    zn = z / znrm_ref[...]
    c = jax.lax.dot_general(zn, wn_ref[...], (((1,), (1,)), ((), ())),
                            preferred_element_type=jnp.float32)
    m = jnp.max(c, axis=1, keepdims=True)
    colidx = jax.lax.broadcasted_iota(jnp.int32, c.shape, 1)
    idx = jnp.min(jnp.where(c == m, colidx, _K), axis=1, keepdims=True)
    e = jnp.exp(c * (1.0 / _TEMP)).astype(jnp.bfloat16)
    ones_k = jnp.ones((_K, 1), jnp.bfloat16)
    s = jax.lax.dot_general(e, ones_k, (((1,), (0,)), ((), ())),
                            preferred_element_type=jnp.float32)
    psum_ref[...] += jax.lax.dot_general(
        (1.0 / s).astype(jnp.bfloat16), e, (((0,), (0,)), ((), ())),
        preferred_element_type=jnp.float32)
    oh = (colidx == idx).astype(jnp.float32)
    ones_n = jnp.ones((oh.shape[0], 1), jnp.float32)
    cnt_ref[...] += jax.lax.dot_general(
        ones_n, oh, (((0,), (0,)), ((), ())),
        preferred_element_type=jnp.float32)
    zq = jax.lax.dot_general(oh, w_ref[...], (((1,), (0,)), ((), ())),
                             preferred_element_type=jnp.float32)
    zq_ref[...] = zq
    diff = zq - z
    com_acc[...] += jnp.sum(diff * diff).reshape(1, 1)

    @pl.when(r == rb - 1)
    def _finalize():
        pavg = psum_ref[...] / n_rows + 1e-8
        ent_ref[...] = -jnp.sum(pavg * jnp.log(pavg)).reshape(1, 1)
        e_mean = cnt_ref[...] / n_rows
        ppl_ref[...] = jnp.exp(
            -jnp.sum(e_mean * jnp.log(e_mean + 1e-8))).reshape(1, 1)
        com_ref[...] = (1.0 + _BETA) * com_acc[...] / (n_rows * _D)


@jax.jit
def _cos_vq(z_flat, znrm, W, wnrm):
    n = z_flat.shape[0]
    rb = n // _NB
    zq, com, ppl, ent = pl.pallas_call(
        functools.partial(_vq_body, n_rows=n, rb=rb),
        grid=(rb,),
        in_specs=[
            pl.BlockSpec((_NB, _D), lambda r: (r, 0)),
            pl.BlockSpec((_NB, 1), lambda r: (r, 0)),
            pl.BlockSpec((_K, _D), lambda r: (0, 0)),
            pl.BlockSpec((_K, 1), lambda r: (0, 0)),
        ],
        out_specs=[
            pl.BlockSpec((_NB, _D), lambda r: (r, 0)),
            pl.BlockSpec((1, 1), lambda r: (0, 0)),
            pl.BlockSpec((1, 1), lambda r: (0, 0)),
            pl.BlockSpec((1, 1), lambda r: (0, 0)),
        ],
        out_shape=[
            jax.ShapeDtypeStruct((n, _D), jnp.float32),
            jax.ShapeDtypeStruct((1, 1), jnp.float32),
            jax.ShapeDtypeStruct((1, 1), jnp.float32),
            jax.ShapeDtypeStruct((1, 1), jnp.float32),
        ],
        scratch_shapes=[
            pltpu.VMEM((_K, _D), jnp.float32),
            pltpu.VMEM((1, _K), jnp.float32),
            pltpu.VMEM((1, _K), jnp.float32),
            pltpu.VMEM((1, 1), jnp.float32),
        ],
    )(z_flat, znrm, W, wnrm)
    return zq, com[0, 0], ppl[0, 0], ent[0, 0]


def kernel(z, W):
    z_flat = z.reshape(-1, _D)
    znrm = jnp.maximum(jnp.linalg.norm(z_flat, axis=1, keepdims=True), 1e-12)
    wnrm = jnp.maximum(jnp.linalg.norm(W, axis=1, keepdims=True), 1e-12)
    zq, com, ppl, ent = _cos_vq(z_flat, znrm, W, wnrm)
    return zq.reshape(z.shape), com, ppl, ent


# speedup vs baseline: 1.0041x; 1.0041x over previous
import functools

import jax
import jax.numpy as jnp
from jax.experimental import pallas as pl
from jax.experimental.pallas import tpu as pltpu

_K = 8192
_D = 128
_BETA = 0.25
_TEMP = 0.1
_NB = 576


def _vq_body(z_ref, znrm_ref, w_ref, wnrm_ref,
             zq_ref, com_ref, ppl_ref, ent_ref,
             wn_ref, psum_ref, cnt_ref, com_acc, n_rows, rb):
    r = pl.program_id(0)

    @pl.when(r == 0)
    def _init():
        wn_ref[...] = w_ref[...] / wnrm_ref[...]
        psum_ref[...] = jnp.zeros_like(psum_ref)
        cnt_ref[...] = jnp.zeros_like(cnt_ref)
        com_acc[...] = jnp.zeros_like(com_acc)

    z = z_ref[...]
    zn = z / znrm_ref[...]
    c = jax.lax.dot_general(zn, wn_ref[...], (((1,), (1,)), ((), ())),
                            preferred_element_type=jnp.float32)
    m = jnp.max(c, axis=1, keepdims=True)
    colidx = jax.lax.broadcasted_iota(jnp.int32, c.shape, 1)
    idx = jnp.min(jnp.where(c == m, colidx, _K), axis=1, keepdims=True)
    e = jnp.exp(c * (1.0 / _TEMP)).astype(jnp.bfloat16)
    ones_k = jnp.ones((_K, 1), jnp.bfloat16)
    s = jax.lax.dot_general(e, ones_k, (((1,), (0,)), ((), ())),
                            preferred_element_type=jnp.float32)
    psum_ref[...] += jax.lax.dot_general(
        (1.0 / s).astype(jnp.bfloat16), e, (((0,), (0,)), ((), ())),
        preferred_element_type=jnp.float32)
    oh = (colidx == idx).astype(jnp.float32)
    ones_n = jnp.ones((oh.shape[0], 1), jnp.float32)
    cnt_ref[...] += jax.lax.dot_general(
        ones_n, oh, (((0,), (0,)), ((), ())),
        preferred_element_type=jnp.float32)
    zq = jax.lax.dot_general(oh, w_ref[...], (((1,), (0,)), ((), ())),
                             preferred_element_type=jnp.float32)
    zq_ref[...] = zq
    diff = zq - z
    com_acc[...] += jnp.sum(diff * diff).reshape(1, 1)

    @pl.when(r == rb - 1)
    def _finalize():
        pavg = psum_ref[...] / n_rows + 1e-8
        ent_ref[...] = -jnp.sum(pavg * jnp.log(pavg)).reshape(1, 1)
        e_mean = cnt_ref[...] / n_rows
        ppl_ref[...] = jnp.exp(
            -jnp.sum(e_mean * jnp.log(e_mean + 1e-8))).reshape(1, 1)
        com_ref[...] = (1.0 + _BETA) * com_acc[...] / (n_rows * _D)


@jax.jit
def _cos_vq(z_flat, znrm, W, wnrm):
    n = z_flat.shape[0]
    rb = n // _NB
    zq, com, ppl, ent = pl.pallas_call(
        functools.partial(_vq_body, n_rows=n, rb=rb),
        grid=(rb,),
        in_specs=[
            pl.BlockSpec((_NB, _D), lambda r: (r, 0)),
            pl.BlockSpec((_NB, 1), lambda r: (r, 0)),
            pl.BlockSpec((_K, _D), lambda r: (0, 0)),
            pl.BlockSpec((_K, 1), lambda r: (0, 0)),
        ],
        out_specs=[
            pl.BlockSpec((_NB, _D), lambda r: (r, 0)),
            pl.BlockSpec((1, 1), lambda r: (0, 0)),
            pl.BlockSpec((1, 1), lambda r: (0, 0)),
            pl.BlockSpec((1, 1), lambda r: (0, 0)),
        ],
        out_shape=[
            jax.ShapeDtypeStruct((n, _D), jnp.float32),
            jax.ShapeDtypeStruct((1, 1), jnp.float32),
            jax.ShapeDtypeStruct((1, 1), jnp.float32),
            jax.ShapeDtypeStruct((1, 1), jnp.float32),
        ],
        scratch_shapes=[
            pltpu.VMEM((_K, _D), jnp.float32),
            pltpu.VMEM((1, _K), jnp.float32),
            pltpu.VMEM((1, _K), jnp.float32),
            pltpu.VMEM((1, 1), jnp.float32),
        ],
    )(z_flat, znrm, W, wnrm)
    return zq, com[0, 0], ppl[0, 0], ent[0, 0]


def kernel(z, W):
    z_flat = z.reshape(-1, _D)
    znrm = jnp.maximum(jnp.linalg.norm(z_flat, axis=1, keepdims=True), 1e-12)
    wnrm = jnp.maximum(jnp.linalg.norm(W, axis=1, keepdims=True), 1e-12)
    zq, com, ppl, ent = _cos_vq(z_flat, znrm, W, wnrm)
    return zq.reshape(z.shape), com, ppl, ent
